# baseline (speedup 1.0000x reference)
import functools

import jax
import jax.numpy as jnp
from jax import lax
from jax.experimental import pallas as pl
from jax.experimental.pallas import tpu as pltpu
from jax.experimental.pallas import tpu_sc as plsc

BATCH = 16384
USER_DIM = 32
ITEM_DIM = 64
OUT_DIM = USER_DIM + ITEM_DIM
NUM_USERS = 1000000

_NC = 2
_NS = 16
_NW = _NC * _NS
_BPW = BATCH // _NW
_NCHUNK = _BPW // 16
_UROW = NUM_USERS // 16


def _make_kernel():
    mesh = plsc.VectorSubcoreMesh(core_axis_name="c", subcore_axis_name="s")

    @functools.partial(
        pl.kernel,
        mesh=mesh,
        out_type=jax.ShapeDtypeStruct((BATCH, OUT_DIM), jnp.float32),
        compiler_params=pltpu.CompilerParams(use_tc_tiling_on_sc=False,
                                             needs_layout_passes=False),
        scratch_types=[
            pltpu.VMEM((_BPW,), jnp.int32),
            pltpu.VMEM((_BPW,), jnp.int32),
            pltpu.VMEM((_BPW,), jnp.int32),
            pltpu.VMEM((_BPW,), jnp.int32),
            pltpu.VMEM((_BPW,), jnp.int32),
            pltpu.VMEM((_BPW, 16), jnp.float32),
            pltpu.VMEM((_BPW, USER_DIM), jnp.float32),
            pltpu.VMEM((_BPW, ITEM_DIM), jnp.float32),
            pltpu.SemaphoreType.DMA,
            pltpu.SemaphoreType.DMA,
        ],
    )
    def gather_concat(user_idx_hbm, item_idx_hbm, uview_hbm, item_tbl_hbm,
                      out_hbm, uidx_v, iidx_v, ubase_v, ulane_v, ridx_v,
                      fdata_v, urows_v, irows_v, usem, isem):
        wid = lax.axis_index("s") * _NC + lax.axis_index("c")
        base = wid * _BPW
        pltpu.sync_copy(user_idx_hbm.at[pl.ds(base, _BPW)], uidx_v)
        pltpu.sync_copy(item_idx_hbm.at[pl.ds(base, _BPW)], iidx_v)

        icp = pltpu.async_copy(item_tbl_hbm.at[iidx_v], irows_v, isem)

        def prep(c, carry):
            u = uidx_v[pl.ds(c * 16, 16)]
            ubase_v[pl.ds(c * 16, 16)] = lax.shift_right_logical(u, 4)
            ulane_v[pl.ds(c * 16, 16)] = lax.bitwise_and(u, 15)
            return carry

        lax.fori_loop(0, _NCHUNK, prep, 0, unroll=4)

        pos_iota = lax.iota(jnp.int32, 16)

        def per_feature(f, carry):
            off = f * _UROW

            def build(c, carry2):
                ridx_v[pl.ds(c * 16, 16)] = ubase_v[pl.ds(c * 16, 16)] + off
                return carry2

            lax.fori_loop(0, _NCHUNK, build, 0, unroll=4)
            pltpu.async_copy(uview_hbm.at[ridx_v], fdata_v, usem).wait()

            fvec = pos_iota * 0 + f

            def extract(c, carry2):
                p = c * 16 + pos_iota
                lanes = ulane_v[pl.ds(c * 16, 16)]
                vals = plsc.load_gather(fdata_v, [p, lanes])
                plsc.store_scatter(urows_v, [p, fvec], vals)
                return carry2

            lax.fori_loop(0, _NCHUNK, extract, 0, unroll=4)
            return carry

        lax.fori_loop(0, USER_DIM, per_feature, 0)

        pltpu.sync_copy(urows_v,
                        out_hbm.at[pl.ds(base, _BPW), pl.ds(0, USER_DIM)])
        icp.wait()
        pltpu.sync_copy(irows_v,
                        out_hbm.at[pl.ds(base, _BPW), pl.ds(USER_DIM, ITEM_DIM)])

    return gather_concat


_gather_concat = _make_kernel()


def kernel(user_input, item_input, user_table, item_table):
    uview = user_table.T.reshape(NUM_USERS // 16 * USER_DIM, 16)
    return _gather_concat(user_input.astype(jnp.int32),
                          item_input.astype(jnp.int32),
                          uview, item_table)


# speedup vs baseline: 4.7106x; 4.7106x over previous
import functools

import jax
import jax.numpy as jnp
from jax import lax
from jax.experimental import pallas as pl
from jax.experimental.pallas import tpu as pltpu
from jax.experimental.pallas import tpu_sc as plsc

BATCH = 16384
USER_DIM = 32
ITEM_DIM = 64
OUT_DIM = USER_DIM + ITEM_DIM

_NC = 2
_NS = 16
_NW = _NC * _NS
_BPW = BATCH // _NW


def _make_kernel():
    mesh = plsc.VectorSubcoreMesh(core_axis_name="c", subcore_axis_name="s")

    @functools.partial(
        pl.kernel,
        mesh=mesh,
        out_type=jax.ShapeDtypeStruct((BATCH, OUT_DIM), jnp.float32),
        compiler_params=pltpu.CompilerParams(use_tc_tiling_on_sc=False),
        scratch_types=[
            pltpu.VMEM((_BPW,), jnp.int32),
            pltpu.VMEM((_BPW,), jnp.int32),
            pltpu.VMEM((_BPW, USER_DIM), jnp.float32),
            pltpu.VMEM((_BPW, ITEM_DIM), jnp.float32),
            pltpu.SemaphoreType.DMA,
            pltpu.SemaphoreType.DMA,
        ],
    )
    def gather_concat(user_idx_hbm, item_idx_hbm, user_tbl_hbm, item_tbl_hbm,
                      out_hbm, uidx_v, iidx_v, urows_v, irows_v, usem, isem):
        wid = lax.axis_index("s") * _NC + lax.axis_index("c")
        base = wid * _BPW
        pltpu.sync_copy(user_idx_hbm.at[pl.ds(base, _BPW)], uidx_v)
        pltpu.sync_copy(item_idx_hbm.at[pl.ds(base, _BPW)], iidx_v)
        ucp = pltpu.async_copy(user_tbl_hbm.at[uidx_v], urows_v, usem)
        icp = pltpu.async_copy(item_tbl_hbm.at[iidx_v], irows_v, isem)
        ucp.wait()
        pltpu.sync_copy(urows_v,
                        out_hbm.at[pl.ds(base, _BPW), pl.ds(0, USER_DIM)])
        icp.wait()
        pltpu.sync_copy(irows_v,
                        out_hbm.at[pl.ds(base, _BPW), pl.ds(USER_DIM, ITEM_DIM)])

    return gather_concat


_gather_concat = _make_kernel()


def kernel(user_input, item_input, user_table, item_table):
    return _gather_concat(user_input.astype(jnp.int32),
                          item_input.astype(jnp.int32),
                          user_table, item_table)
